# Initial kernel scaffold; baseline (speedup 1.0000x reference)
import jax
import jax.numpy as jnp
from jax.experimental import pallas as pl


def kernel(x, feat, edge_index, W1, b1, W2, b2, Wfc, bfc):
    raise NotImplementedError("write your pallas kernel here")


# speedup vs baseline: 13.6939x; 13.6939x over previous
import functools

import jax
import jax.numpy as jnp
from jax import lax
from jax.experimental import pallas as pl
from jax.experimental.pallas import tpu as pltpu
from jax.experimental.pallas import tpu_sc as plsc

N = 50000
E = 800000
H = 32
NSUB = 6400
EPAD = NSUB * 128
ACC_ROWS = N + 128
RPT = 3128
BLK = 1000
GRID = N // BLK

_mesh = plsc.VectorSubcoreMesh(core_axis_name="c", subcore_axis_name="s")
_sc_params = pltpu.CompilerParams(use_tc_tiling_on_sc=False)


def _deg_body(row_hbm, cnt_hbm, acc, ibuf, ones, zbuf, dbuf):
    c = lax.axis_index("c")
    s = lax.axis_index("s")
    w = s * 2 + c

    def fill(i, _):
        ones[pl.ds(i * 16, 16)] = jnp.ones((16,), jnp.float32)
        return 0
    lax.fori_loop(0, 8, fill, 0)

    def zfill(i, _):
        zbuf[pl.ds(i * 16, 16)] = jnp.zeros((16,), jnp.float32)
        return 0
    lax.fori_loop(0, 201, zfill, 0)
    pltpu.sync_copy(zbuf.at[pl.ds(0, RPT)], acc.at[pl.ds(s * RPT, RPT)])

    @pl.when(s == 15)
    def _():
        pltpu.sync_copy(zbuf.at[pl.ds(0, 80)], acc.at[pl.ds(16 * RPT, 80)])
    plsc.subcore_barrier()

    def step(g, _):
        sub0 = w * 200 + g * 8
        pltpu.sync_copy(row_hbm.at[pl.ds(sub0, 8)], ibuf)
        for j in range(8):
            pltpu.sync_copy(ones, acc.at[ibuf.at[j]], add=True)
        return 0
    lax.fori_loop(0, 25, step, 0)
    plsc.subcore_barrier()
    pltpu.sync_copy(acc.at[pl.ds(s * RPT, RPT)], dbuf)
    pltpu.sync_copy(dbuf, cnt_hbm.at[pl.ds(c * ACC_ROWS + s * RPT, RPT)])


_deg_call = functools.partial(
    pl.kernel,
    out_type=jax.ShapeDtypeStruct((2 * ACC_ROWS,), jnp.float32),
    mesh=_mesh,
    scratch_types=[
        pltpu.VMEM_SHARED((ACC_ROWS,), jnp.float32),
        pltpu.VMEM((8, 128), jnp.int32),
        pltpu.VMEM((128,), jnp.float32),
        pltpu.VMEM((3216,), jnp.float32),
        pltpu.VMEM((RPT,), jnp.float32),
    ],
    compiler_params=_sc_params,
)(_deg_body)


KP = 4
STEPS = 400 // KP


def _prop_body(u_hbm, rows2_hbm, col_hbm, s_hbm, acc, rbuf, cbuf, mbuf, sem):
    c = lax.axis_index("c")
    s = lax.axis_index("s")
    zsrc = mbuf.at[0]

    def zb(i, _):
        mbuf[0, i, 0:16] = jnp.zeros((16,), jnp.float32)
        mbuf[0, i, 16:32] = jnp.zeros((16,), jnp.float32)
        return 0
    lax.fori_loop(0, 128, zb, 0)

    def zc(k, _):
        pltpu.sync_copy(zsrc, acc.at[pl.ds(s * RPT + k * 128, 128)])
        return 0
    lax.fori_loop(0, 24, zc, 0)
    pltpu.sync_copy(zsrc.at[pl.ds(0, 56)],
                    acc.at[pl.ds(s * RPT + 24 * 128, 56)])

    @pl.when(s == 15)
    def _():
        pltpu.sync_copy(zsrc.at[pl.ds(0, 80)], acc.at[pl.ds(16 * RPT, 80)])
    plsc.subcore_barrier()

    def step(g, _):
        sub0 = s * 400 + g * KP
        pltpu.sync_copy(rows2_hbm.at[pl.ds(c * NSUB + sub0, KP)], rbuf)
        pltpu.sync_copy(col_hbm.at[pl.ds(sub0, KP)], cbuf)
        descs = [pltpu.async_copy(u_hbm.at[rbuf.at[j]], mbuf.at[j], sem)
                 for j in range(KP)]
        for d in descs:
            d.wait()
        for j in range(KP):
            pltpu.sync_copy(mbuf.at[j], acc.at[cbuf.at[j]], add=True)
        return 0
    lax.fori_loop(0, STEPS, step, 0)
    plsc.subcore_barrier()

    bounce = mbuf.at[0]

    def cp(k, _):
        pltpu.sync_copy(acc.at[pl.ds(s * RPT + k * 128, 128)], bounce)
        pltpu.sync_copy(bounce, s_hbm.at[pl.ds(c * N + s * RPT + k * 128, 128)])
        return 0
    lax.fori_loop(0, 24, cp, 0)

    @pl.when(s < 15)
    def _():
        pltpu.sync_copy(acc.at[pl.ds(s * RPT + 3072, 56)],
                        bounce.at[pl.ds(0, 56)])
        pltpu.sync_copy(bounce.at[pl.ds(0, 56)],
                        s_hbm.at[pl.ds(c * N + s * RPT + 3072, 56)])

    @pl.when(s == 15)
    def _():
        pltpu.sync_copy(acc.at[pl.ds(15 * RPT + 3072, 8)],
                        bounce.at[pl.ds(0, 8)])
        pltpu.sync_copy(bounce.at[pl.ds(0, 8)],
                        s_hbm.at[pl.ds(c * N + 15 * RPT + 3072, 8)])


_prop_call = functools.partial(
    pl.kernel,
    out_type=jax.ShapeDtypeStruct((2 * N, H), jnp.float32),
    mesh=_mesh,
    scratch_types=[
        pltpu.VMEM_SHARED((ACC_ROWS, H), jnp.float32),
        pltpu.VMEM((KP, 128), jnp.int32),
        pltpu.VMEM((KP, 128), jnp.int32),
        pltpu.VMEM((KP, 128, H), jnp.float32),
        pltpu.SemaphoreType.DMA,
    ],
    compiler_params=_sc_params,
)(_prop_body)


def _prep_kernel(row_ref, out_ref):
    c = pl.program_id(0)
    v = row_ref[...]
    out_ref[...] = jnp.where(v >= N, 0, v + c * N)


def _prep_call(rowp):
    return pl.pallas_call(
        _prep_kernel,
        grid=(2, NSUB // 128),
        in_specs=[pl.BlockSpec((128, 128), lambda c, i: (i, 0))],
        out_specs=pl.BlockSpec((128, 128), lambda c, i: (c * (NSUB // 128) + i, 0)),
        out_shape=jax.ShapeDtypeStruct((2 * NSUB // 128 * 128, 128), jnp.int32),
    )(rowp)


def _dis_of(cnt_ref):
    deg = cnt_ref[0, :, 0:1] + cnt_ref[1, :, 0:1] + 1.0
    return lax.rsqrt(deg)


def _tc1_kernel(x_ref, f_ref, cnt_ref, wa_ref, wb_ref, ba_ref, bb_ref, u_ref):
    dis = _dis_of(cnt_ref)
    x = x_ref[...]
    f = f_ref[...]
    ta = (jnp.dot(x, wa_ref[0:64, :], preferred_element_type=jnp.float32)
          + jnp.dot(f, wa_ref[64:128, :], preferred_element_type=jnp.float32)
          + ba_ref[...])
    tb = (jnp.dot(x, wb_ref[0:64, :], preferred_element_type=jnp.float32)
          + jnp.dot(f, wb_ref[64:128, :], preferred_element_type=jnp.float32)
          + bb_ref[...])
    u_ref[0] = dis * ta
    u_ref[1] = dis * tb


def _tc1_call(x, feat, cnt3, wa, wb, ba, bb):
    full = lambda shape: pl.BlockSpec(shape, lambda i: tuple(0 for _ in shape))
    return pl.pallas_call(
        _tc1_kernel,
        grid=(GRID,),
        in_specs=[
            pl.BlockSpec((BLK, 64), lambda i: (i, 0)),
            pl.BlockSpec((BLK, 64), lambda i: (i, 0)),
            pl.BlockSpec((2, BLK, 1), lambda i: (0, i, 0)),
            full((128, H)), full((128, H)), full((1, H)), full((1, H)),
        ],
        out_specs=pl.BlockSpec((2, BLK, H), lambda i: (0, i, 0)),
        out_shape=jax.ShapeDtypeStruct((2, N, H), jnp.float32),
    )(x, feat, cnt3, wa, wb, ba, bb)


def _tc2_kernel(s_ref, u_ref, cnt_ref, wa_ref, wb_ref, ba_ref, bb_ref,
                uo_ref):
    dis = _dis_of(cnt_ref)
    ha = jnp.maximum(dis * (s_ref[0] + u_ref[0]), 0.0)
    hb = jnp.maximum(dis * (s_ref[1] + u_ref[1]), 0.0)
    ta = (jnp.dot(ha, wa_ref[0:H, :], preferred_element_type=jnp.float32)
          + jnp.dot(hb, wa_ref[H:2 * H, :], preferred_element_type=jnp.float32)
          + ba_ref[...])
    tb = (jnp.dot(ha, wb_ref[0:H, :], preferred_element_type=jnp.float32)
          + jnp.dot(hb, wb_ref[H:2 * H, :], preferred_element_type=jnp.float32)
          + bb_ref[...])
    uo_ref[0] = dis * ta
    uo_ref[1] = dis * tb


def _tc2_call(s1, u1, cnt3, wa, wb, ba, bb):
    full = lambda shape: pl.BlockSpec(shape, lambda i: tuple(0 for _ in shape))
    return pl.pallas_call(
        _tc2_kernel,
        grid=(GRID,),
        in_specs=[
            pl.BlockSpec((2, BLK, H), lambda i: (0, i, 0)),
            pl.BlockSpec((2, BLK, H), lambda i: (0, i, 0)),
            pl.BlockSpec((2, BLK, 1), lambda i: (0, i, 0)),
            full((2 * H, H)), full((2 * H, H)), full((1, H)), full((1, H)),
        ],
        out_specs=pl.BlockSpec((2, BLK, H), lambda i: (0, i, 0)),
        out_shape=jax.ShapeDtypeStruct((2, N, H), jnp.float32),
    )(s1, u1, cnt3, wa, wb, ba, bb)


def _tc3_kernel(s_ref, u_ref, cnt_ref, w_ref, b_ref, o_ref):
    dis = _dis_of(cnt_ref)
    ha = jnp.maximum(dis * (s_ref[0] + u_ref[0]), 0.0)
    hb = jnp.maximum(dis * (s_ref[1] + u_ref[1]), 0.0)
    t = (jnp.dot(ha, w_ref[0:H, :], preferred_element_type=jnp.float32)
         + jnp.dot(hb, w_ref[H:2 * H, :], preferred_element_type=jnp.float32)
         + b_ref[...])
    o_ref[...] = jnp.maximum(t, 0.0)


def _tc3_call(s2, u2, cnt3, w, b):
    full = lambda shape: pl.BlockSpec(shape, lambda i: tuple(0 for _ in shape))
    return pl.pallas_call(
        _tc3_kernel,
        grid=(GRID,),
        in_specs=[
            pl.BlockSpec((2, BLK, H), lambda i: (0, i, 0)),
            pl.BlockSpec((2, BLK, H), lambda i: (0, i, 0)),
            pl.BlockSpec((2, BLK, 1), lambda i: (0, i, 0)),
            full((2 * H, 64)), full((1, 64)),
        ],
        out_specs=pl.BlockSpec((BLK, 64), lambda i: (i, 0)),
        out_shape=jax.ShapeDtypeStruct((N, 64), jnp.float32),
    )(s2, u2, cnt3, w, b)


def kernel(x, feat, edge_index, W1, b1, W2, b2, Wfc, bfc):
    row = edge_index[0]
    col = edge_index[1]
    pad = jnp.full((EPAD - E,), N, jnp.int32)
    rowp = jnp.concatenate([row, pad]).reshape(NSUB, 128)
    colp = jnp.concatenate([col, pad]).reshape(NSUB, 128)

    w1t = W1.T
    wa1, wb1 = w1t[:, :H], w1t[:, H:]
    ba1, bb1 = b1[:H].reshape(1, H), b1[H:].reshape(1, H)
    w2t = W2.T
    wa2, wb2 = w2t[:, :H], w2t[:, H:]
    ba2, bb2 = b2[:H].reshape(1, H), b2[H:].reshape(1, H)
    wfct = Wfc.T
    bfc2 = bfc.reshape(1, 64)

    cnt = _deg_call(rowp)
    cnt3 = cnt.reshape(2, ACC_ROWS, 1)
    rows2 = _prep_call(rowp)

    u1 = _tc1_call(x, feat, cnt3, wa1, wb1, ba1, bb1)
    s1 = _prop_call(u1.reshape(2 * N, H), rows2, colp)
    u2 = _tc2_call(s1.reshape(2, N, H), u1, cnt3, wa2, wb2, ba2, bb2)
    s2 = _prop_call(u2.reshape(2 * N, H), rows2, colp)
    return _tc3_call(s2.reshape(2, N, H), u2, cnt3, wfct, bfc2)


# speedup vs baseline: 14.1699x; 1.0348x over previous
import functools

import jax
import jax.numpy as jnp
from jax import lax
from jax.experimental import pallas as pl
from jax.experimental.pallas import tpu as pltpu
from jax.experimental.pallas import tpu_sc as plsc

N = 50000
E = 800000
H = 32
NSUB = 6400
EPAD = NSUB * 128
ACC_ROWS = N + 128
RPT = 3128
BLK = 1000
GRID = N // BLK

_mesh = plsc.VectorSubcoreMesh(core_axis_name="c", subcore_axis_name="s")
_sc_params = pltpu.CompilerParams(use_tc_tiling_on_sc=False)


def _deg_body(row_hbm, cnt_hbm, acc, ibuf, ones, zbuf, dbuf):
    c = lax.axis_index("c")
    s = lax.axis_index("s")
    w = s * 2 + c

    def fill(i, _):
        ones[pl.ds(i * 16, 16)] = jnp.ones((16,), jnp.float32)
        return 0
    lax.fori_loop(0, 8, fill, 0)

    def zfill(i, _):
        zbuf[pl.ds(i * 16, 16)] = jnp.zeros((16,), jnp.float32)
        return 0
    lax.fori_loop(0, 201, zfill, 0)
    pltpu.sync_copy(zbuf.at[pl.ds(0, RPT)], acc.at[pl.ds(s * RPT, RPT)])

    @pl.when(s == 15)
    def _():
        pltpu.sync_copy(zbuf.at[pl.ds(0, 80)], acc.at[pl.ds(16 * RPT, 80)])
    plsc.subcore_barrier()

    def step(g, _):
        sub0 = w * 200 + g * 8
        pltpu.sync_copy(row_hbm.at[pl.ds(sub0, 8)], ibuf)
        for j in range(8):
            pltpu.sync_copy(ones, acc.at[ibuf.at[j]], add=True)
        return 0
    lax.fori_loop(0, 25, step, 0)
    plsc.subcore_barrier()
    pltpu.sync_copy(acc.at[pl.ds(s * RPT, RPT)], dbuf)
    pltpu.sync_copy(dbuf, cnt_hbm.at[pl.ds(c * ACC_ROWS + s * RPT, RPT)])


_deg_call = functools.partial(
    pl.kernel,
    out_type=jax.ShapeDtypeStruct((2 * ACC_ROWS,), jnp.float32),
    mesh=_mesh,
    scratch_types=[
        pltpu.VMEM_SHARED((ACC_ROWS,), jnp.float32),
        pltpu.VMEM((8, 128), jnp.int32),
        pltpu.VMEM((128,), jnp.float32),
        pltpu.VMEM((3216,), jnp.float32),
        pltpu.VMEM((RPT,), jnp.float32),
    ],
    compiler_params=_sc_params,
)(_deg_body)


KB = 2
NBATCH = 400 // KB


def _prop_body(u_hbm, rows2_hbm, col_hbm, s_hbm, acc, rbuf, cbuf, mbuf,
               sem0, sem1):
    c = lax.axis_index("c")
    s = lax.axis_index("s")
    sems = (sem0, sem1)
    zsrc = mbuf.at[0, 0]

    def zb(i, _):
        mbuf[0, 0, i, 0:16] = jnp.zeros((16,), jnp.float32)
        mbuf[0, 0, i, 16:32] = jnp.zeros((16,), jnp.float32)
        return 0
    lax.fori_loop(0, 128, zb, 0)

    def zc(k, _):
        pltpu.sync_copy(zsrc, acc.at[pl.ds(s * RPT + k * 128, 128)])
        return 0
    lax.fori_loop(0, 24, zc, 0)
    pltpu.sync_copy(zsrc.at[pl.ds(0, 56)],
                    acc.at[pl.ds(s * RPT + 24 * 128, 56)])

    @pl.when(s == 15)
    def _():
        pltpu.sync_copy(zsrc.at[pl.ds(0, 80)], acc.at[pl.ds(16 * RPT, 80)])
    plsc.subcore_barrier()

    def issue(sel, batch):
        sub0 = s * 400 + batch * KB
        pltpu.sync_copy(rows2_hbm.at[pl.ds(c * NSUB + sub0, KB)],
                        rbuf.at[sel])
        pltpu.sync_copy(col_hbm.at[pl.ds(sub0, KB)], cbuf.at[sel])
        for j in range(KB):
            pltpu.async_copy(u_hbm.at[rbuf.at[sel, j]], mbuf.at[sel, j],
                             sems[sel])

    def drain_scatter(sel):
        for j in range(KB):
            pltpu.make_async_copy(u_hbm.at[rbuf.at[sel, j]],
                                  mbuf.at[sel, j], sems[sel]).wait()
        for j in range(KB):
            pltpu.sync_copy(mbuf.at[sel, j], acc.at[cbuf.at[sel, j]],
                            add=True)

    issue(0, 0)

    def step(i, _):
        issue(1, 2 * i + 1)
        drain_scatter(0)

        @pl.when(i < NBATCH // 2 - 1)
        def _():
            issue(0, 2 * i + 2)
        drain_scatter(1)
        return 0
    lax.fori_loop(0, NBATCH // 2, step, 0)
    plsc.subcore_barrier()

    bounce = mbuf.at[0, 0]

    def cp(k, _):
        pltpu.sync_copy(acc.at[pl.ds(s * RPT + k * 128, 128)], bounce)
        pltpu.sync_copy(bounce, s_hbm.at[pl.ds(c * N + s * RPT + k * 128, 128)])
        return 0
    lax.fori_loop(0, 24, cp, 0)

    @pl.when(s < 15)
    def _():
        pltpu.sync_copy(acc.at[pl.ds(s * RPT + 3072, 56)],
                        bounce.at[pl.ds(0, 56)])
        pltpu.sync_copy(bounce.at[pl.ds(0, 56)],
                        s_hbm.at[pl.ds(c * N + s * RPT + 3072, 56)])

    @pl.when(s == 15)
    def _():
        pltpu.sync_copy(acc.at[pl.ds(15 * RPT + 3072, 8)],
                        bounce.at[pl.ds(0, 8)])
        pltpu.sync_copy(bounce.at[pl.ds(0, 8)],
                        s_hbm.at[pl.ds(c * N + 15 * RPT + 3072, 8)])


_prop_call = functools.partial(
    pl.kernel,
    out_type=jax.ShapeDtypeStruct((2 * N, H), jnp.float32),
    mesh=_mesh,
    scratch_types=[
        pltpu.VMEM_SHARED((ACC_ROWS, H), jnp.float32),
        pltpu.VMEM((2, KB, 128), jnp.int32),
        pltpu.VMEM((2, KB, 128), jnp.int32),
        pltpu.VMEM((2, KB, 128, H), jnp.float32),
        pltpu.SemaphoreType.DMA,
        pltpu.SemaphoreType.DMA,
    ],
    compiler_params=_sc_params,
)(_prop_body)


def _prep_kernel(row_ref, out_ref):
    c = pl.program_id(0)
    v = row_ref[...]
    out_ref[...] = jnp.where(v >= N, 0, v + c * N)


def _prep_call(rowp):
    return pl.pallas_call(
        _prep_kernel,
        grid=(2, NSUB // 128),
        in_specs=[pl.BlockSpec((128, 128), lambda c, i: (i, 0))],
        out_specs=pl.BlockSpec((128, 128), lambda c, i: (c * (NSUB // 128) + i, 0)),
        out_shape=jax.ShapeDtypeStruct((2 * NSUB // 128 * 128, 128), jnp.int32),
    )(rowp)


def _dis_of(cnt_ref):
    deg = cnt_ref[0, :, 0:1] + cnt_ref[1, :, 0:1] + 1.0
    return lax.rsqrt(deg)


def _tc1_kernel(x_ref, f_ref, cnt_ref, wa_ref, wb_ref, ba_ref, bb_ref, u_ref):
    dis = _dis_of(cnt_ref)
    x = x_ref[...]
    f = f_ref[...]
    ta = (jnp.dot(x, wa_ref[0:64, :], preferred_element_type=jnp.float32)
          + jnp.dot(f, wa_ref[64:128, :], preferred_element_type=jnp.float32)
          + ba_ref[...])
    tb = (jnp.dot(x, wb_ref[0:64, :], preferred_element_type=jnp.float32)
          + jnp.dot(f, wb_ref[64:128, :], preferred_element_type=jnp.float32)
          + bb_ref[...])
    u_ref[0] = dis * ta
    u_ref[1] = dis * tb


def _tc1_call(x, feat, cnt3, wa, wb, ba, bb):
    full = lambda shape: pl.BlockSpec(shape, lambda i: tuple(0 for _ in shape))
    return pl.pallas_call(
        _tc1_kernel,
        grid=(GRID,),
        in_specs=[
            pl.BlockSpec((BLK, 64), lambda i: (i, 0)),
            pl.BlockSpec((BLK, 64), lambda i: (i, 0)),
            pl.BlockSpec((2, BLK, 1), lambda i: (0, i, 0)),
            full((128, H)), full((128, H)), full((1, H)), full((1, H)),
        ],
        out_specs=pl.BlockSpec((2, BLK, H), lambda i: (0, i, 0)),
        out_shape=jax.ShapeDtypeStruct((2, N, H), jnp.float32),
    )(x, feat, cnt3, wa, wb, ba, bb)


def _tc2_kernel(s_ref, u_ref, cnt_ref, wa_ref, wb_ref, ba_ref, bb_ref,
                uo_ref):
    dis = _dis_of(cnt_ref)
    ha = jnp.maximum(dis * (s_ref[0] + u_ref[0]), 0.0)
    hb = jnp.maximum(dis * (s_ref[1] + u_ref[1]), 0.0)
    ta = (jnp.dot(ha, wa_ref[0:H, :], preferred_element_type=jnp.float32)
          + jnp.dot(hb, wa_ref[H:2 * H, :], preferred_element_type=jnp.float32)
          + ba_ref[...])
    tb = (jnp.dot(ha, wb_ref[0:H, :], preferred_element_type=jnp.float32)
          + jnp.dot(hb, wb_ref[H:2 * H, :], preferred_element_type=jnp.float32)
          + bb_ref[...])
    uo_ref[0] = dis * ta
    uo_ref[1] = dis * tb


def _tc2_call(s1, u1, cnt3, wa, wb, ba, bb):
    full = lambda shape: pl.BlockSpec(shape, lambda i: tuple(0 for _ in shape))
    return pl.pallas_call(
        _tc2_kernel,
        grid=(GRID,),
        in_specs=[
            pl.BlockSpec((2, BLK, H), lambda i: (0, i, 0)),
            pl.BlockSpec((2, BLK, H), lambda i: (0, i, 0)),
            pl.BlockSpec((2, BLK, 1), lambda i: (0, i, 0)),
            full((2 * H, H)), full((2 * H, H)), full((1, H)), full((1, H)),
        ],
        out_specs=pl.BlockSpec((2, BLK, H), lambda i: (0, i, 0)),
        out_shape=jax.ShapeDtypeStruct((2, N, H), jnp.float32),
    )(s1, u1, cnt3, wa, wb, ba, bb)


def _tc3_kernel(s_ref, u_ref, cnt_ref, w_ref, b_ref, o_ref):
    dis = _dis_of(cnt_ref)
    ha = jnp.maximum(dis * (s_ref[0] + u_ref[0]), 0.0)
    hb = jnp.maximum(dis * (s_ref[1] + u_ref[1]), 0.0)
    t = (jnp.dot(ha, w_ref[0:H, :], preferred_element_type=jnp.float32)
         + jnp.dot(hb, w_ref[H:2 * H, :], preferred_element_type=jnp.float32)
         + b_ref[...])
    o_ref[...] = jnp.maximum(t, 0.0)


def _tc3_call(s2, u2, cnt3, w, b):
    full = lambda shape: pl.BlockSpec(shape, lambda i: tuple(0 for _ in shape))
    return pl.pallas_call(
        _tc3_kernel,
        grid=(GRID,),
        in_specs=[
            pl.BlockSpec((2, BLK, H), lambda i: (0, i, 0)),
            pl.BlockSpec((2, BLK, H), lambda i: (0, i, 0)),
            pl.BlockSpec((2, BLK, 1), lambda i: (0, i, 0)),
            full((2 * H, 64)), full((1, 64)),
        ],
        out_specs=pl.BlockSpec((BLK, 64), lambda i: (i, 0)),
        out_shape=jax.ShapeDtypeStruct((N, 64), jnp.float32),
    )(s2, u2, cnt3, w, b)


def kernel(x, feat, edge_index, W1, b1, W2, b2, Wfc, bfc):
    row = edge_index[0]
    col = edge_index[1]
    pad = jnp.full((EPAD - E,), N, jnp.int32)
    rowp = jnp.concatenate([row, pad]).reshape(NSUB, 128)
    colp = jnp.concatenate([col, pad]).reshape(NSUB, 128)

    w1t = W1.T
    wa1, wb1 = w1t[:, :H], w1t[:, H:]
    ba1, bb1 = b1[:H].reshape(1, H), b1[H:].reshape(1, H)
    w2t = W2.T
    wa2, wb2 = w2t[:, :H], w2t[:, H:]
    ba2, bb2 = b2[:H].reshape(1, H), b2[H:].reshape(1, H)
    wfct = Wfc.T
    bfc2 = bfc.reshape(1, 64)

    cnt = _deg_call(rowp)
    cnt3 = cnt.reshape(2, ACC_ROWS, 1)
    rows2 = _prep_call(rowp)

    u1 = _tc1_call(x, feat, cnt3, wa1, wb1, ba1, bb1)
    s1 = _prop_call(u1.reshape(2 * N, H), rows2, colp)
    u2 = _tc2_call(s1.reshape(2, N, H), u1, cnt3, wa2, wb2, ba2, bb2)
    s2 = _prop_call(u2.reshape(2 * N, H), rows2, colp)
    return _tc3_call(s2.reshape(2, N, H), u2, cnt3, wfct, bfc2)


# speedup vs baseline: 15.1474x; 1.0690x over previous
import functools

import jax
import jax.numpy as jnp
from jax import lax
from jax.experimental import pallas as pl
from jax.experimental.pallas import tpu as pltpu
from jax.experimental.pallas import tpu_sc as plsc

N = 50000
E = 800000
H = 32
NSUB = 6400
EPAD = NSUB * 128
ACC_ROWS = N + 128
RPT = 3128
BLK = 1000
GRID = N // BLK

_mesh = plsc.VectorSubcoreMesh(core_axis_name="c", subcore_axis_name="s")
_sc_params = pltpu.CompilerParams(use_tc_tiling_on_sc=False)


def _deg_body(row_hbm, cnt_hbm, acc, ibuf, ones, zbuf, dbuf):
    c = lax.axis_index("c")
    s = lax.axis_index("s")
    w = s * 2 + c

    def fill(i, _):
        ones[pl.ds(i * 16, 16)] = jnp.ones((16,), jnp.float32)
        return 0
    lax.fori_loop(0, 8, fill, 0)

    def zfill(i, _):
        zbuf[pl.ds(i * 16, 16)] = jnp.zeros((16,), jnp.float32)
        return 0
    lax.fori_loop(0, 201, zfill, 0)
    pltpu.sync_copy(zbuf.at[pl.ds(0, RPT)], acc.at[pl.ds(s * RPT, RPT)])

    @pl.when(s == 15)
    def _():
        pltpu.sync_copy(zbuf.at[pl.ds(0, 80)], acc.at[pl.ds(16 * RPT, 80)])
    plsc.subcore_barrier()

    def step(g, _):
        sub0 = w * 200 + g * 8
        pltpu.sync_copy(row_hbm.at[pl.ds(sub0, 8)], ibuf)
        for j in range(8):
            pltpu.sync_copy(ones, acc.at[ibuf.at[j]], add=True)
        return 0
    lax.fori_loop(0, 25, step, 0)
    plsc.subcore_barrier()
    pltpu.sync_copy(acc.at[pl.ds(s * RPT, RPT)], dbuf)
    pltpu.sync_copy(dbuf, cnt_hbm.at[pl.ds(c * ACC_ROWS + s * RPT, RPT)])


_deg_call = functools.partial(
    pl.kernel,
    out_type=jax.ShapeDtypeStruct((2 * ACC_ROWS,), jnp.float32),
    mesh=_mesh,
    scratch_types=[
        pltpu.VMEM_SHARED((ACC_ROWS,), jnp.float32),
        pltpu.VMEM((8, 128), jnp.int32),
        pltpu.VMEM((128,), jnp.float32),
        pltpu.VMEM((3216,), jnp.float32),
        pltpu.VMEM((RPT,), jnp.float32),
    ],
    compiler_params=_sc_params,
)(_deg_body)


NSLOT = 4
IB = 8
NB = 400 // IB


def _prop_body(u_hbm, rows2_hbm, col_hbm, s_hbm, acc, ibufr, ibufc, mbuf,
               sem0, sem1, sem2, sem3):
    c = lax.axis_index("c")
    s = lax.axis_index("s")
    sems = (sem0, sem1, sem2, sem3)
    zsrc = mbuf.at[0]

    def zb(i, _):
        mbuf[0, i, 0:16] = jnp.zeros((16,), jnp.float32)
        mbuf[0, i, 16:32] = jnp.zeros((16,), jnp.float32)
        return 0
    lax.fori_loop(0, 128, zb, 0)

    def zc(k, _):
        pltpu.sync_copy(zsrc, acc.at[pl.ds(s * RPT + k * 128, 128)])
        return 0
    lax.fori_loop(0, 24, zc, 0)
    pltpu.sync_copy(zsrc.at[pl.ds(0, 56)],
                    acc.at[pl.ds(s * RPT + 24 * 128, 56)])

    @pl.when(s == 15)
    def _():
        pltpu.sync_copy(zsrc.at[pl.ds(0, 80)], acc.at[pl.ds(16 * RPT, 80)])
    plsc.subcore_barrier()

    def wait_slot(slot):
        pltpu.make_async_copy(u_hbm.at[ibufr.at[0, 0]], mbuf.at[slot],
                              sems[slot]).wait()

    def load_idx(b, par):
        pltpu.sync_copy(
            rows2_hbm.at[pl.ds(c * NSUB + s * 400 + b * IB, IB)],
            ibufr.at[par])
        pltpu.sync_copy(col_hbm.at[pl.ds(s * 400 + b * IB, IB)],
                        ibufc.at[par])

    def gather(slot, par, j):
        pltpu.async_copy(u_hbm.at[ibufr.at[par, j]], mbuf.at[slot],
                         sems[slot])

    def scatter(slot, par, j):
        pltpu.sync_copy(mbuf.at[slot], acc.at[ibufc.at[par, j]], add=True)

    def batch_body(b, p):
        for j in range(IB):
            slot = j % NSLOT
            gather(slot, p, j)
            if j == 4:
                @pl.when(b < NB - 1)
                def _():
                    load_idx(b + 1, 1 - p)
            jb = j - 2
            slotb = jb % NSLOT
            if jb < 0:
                @pl.when(b >= 1)
                def _():
                    _seq = (wait_slot(slotb), scatter(slotb, 1 - p, jb + IB))
            else:
                wait_slot(slotb)
                scatter(slotb, p, jb)

    load_idx(jnp.int32(0), 0)

    def outer(bb, _):
        batch_body(2 * bb, 0)
        batch_body(2 * bb + 1, 1)
        return 0
    lax.fori_loop(0, NB // 2, outer, 0)

    for j in (6, 7):
        slot = j % NSLOT
        wait_slot(slot)
        scatter(slot, 1, j)
    plsc.subcore_barrier()

    bounce = mbuf.at[0]

    def cp(k, _):
        pltpu.sync_copy(acc.at[pl.ds(s * RPT + k * 128, 128)], bounce)
        pltpu.sync_copy(bounce, s_hbm.at[pl.ds(c * N + s * RPT + k * 128, 128)])
        return 0
    lax.fori_loop(0, 24, cp, 0)

    @pl.when(s < 15)
    def _():
        pltpu.sync_copy(acc.at[pl.ds(s * RPT + 3072, 56)],
                        bounce.at[pl.ds(0, 56)])
        pltpu.sync_copy(bounce.at[pl.ds(0, 56)],
                        s_hbm.at[pl.ds(c * N + s * RPT + 3072, 56)])

    @pl.when(s == 15)
    def _():
        pltpu.sync_copy(acc.at[pl.ds(15 * RPT + 3072, 8)],
                        bounce.at[pl.ds(0, 8)])
        pltpu.sync_copy(bounce.at[pl.ds(0, 8)],
                        s_hbm.at[pl.ds(c * N + 15 * RPT + 3072, 8)])


_prop_call = functools.partial(
    pl.kernel,
    out_type=jax.ShapeDtypeStruct((2 * N, H), jnp.float32),
    mesh=_mesh,
    scratch_types=[
        pltpu.VMEM_SHARED((ACC_ROWS, H), jnp.float32),
        pltpu.VMEM((2, IB, 128), jnp.int32),
        pltpu.VMEM((2, IB, 128), jnp.int32),
        pltpu.VMEM((NSLOT, 128, H), jnp.float32),
        pltpu.SemaphoreType.DMA,
        pltpu.SemaphoreType.DMA,
        pltpu.SemaphoreType.DMA,
        pltpu.SemaphoreType.DMA,
    ],
    compiler_params=_sc_params,
)(_prop_body)


def _prep_kernel(row_ref, out_ref):
    c = pl.program_id(0)
    v = row_ref[...]
    out_ref[...] = jnp.where(v >= N, 0, v + c * N)


def _prep_call(rowp):
    return pl.pallas_call(
        _prep_kernel,
        grid=(2, NSUB // 128),
        in_specs=[pl.BlockSpec((128, 128), lambda c, i: (i, 0))],
        out_specs=pl.BlockSpec((128, 128), lambda c, i: (c * (NSUB // 128) + i, 0)),
        out_shape=jax.ShapeDtypeStruct((2 * NSUB // 128 * 128, 128), jnp.int32),
    )(rowp)


def _dis_of(cnt_ref):
    deg = cnt_ref[0, :, 0:1] + cnt_ref[1, :, 0:1] + 1.0
    return lax.rsqrt(deg)


def _tc1_kernel(x_ref, f_ref, cnt_ref, wa_ref, wb_ref, ba_ref, bb_ref, u_ref):
    dis = _dis_of(cnt_ref)
    x = x_ref[...]
    f = f_ref[...]
    ta = (jnp.dot(x, wa_ref[0:64, :], preferred_element_type=jnp.float32)
          + jnp.dot(f, wa_ref[64:128, :], preferred_element_type=jnp.float32)
          + ba_ref[...])
    tb = (jnp.dot(x, wb_ref[0:64, :], preferred_element_type=jnp.float32)
          + jnp.dot(f, wb_ref[64:128, :], preferred_element_type=jnp.float32)
          + bb_ref[...])
    u_ref[0] = dis * ta
    u_ref[1] = dis * tb


def _tc1_call(x, feat, cnt3, wa, wb, ba, bb):
    full = lambda shape: pl.BlockSpec(shape, lambda i: tuple(0 for _ in shape))
    return pl.pallas_call(
        _tc1_kernel,
        grid=(GRID,),
        in_specs=[
            pl.BlockSpec((BLK, 64), lambda i: (i, 0)),
            pl.BlockSpec((BLK, 64), lambda i: (i, 0)),
            pl.BlockSpec((2, BLK, 1), lambda i: (0, i, 0)),
            full((128, H)), full((128, H)), full((1, H)), full((1, H)),
        ],
        out_specs=pl.BlockSpec((2, BLK, H), lambda i: (0, i, 0)),
        out_shape=jax.ShapeDtypeStruct((2, N, H), jnp.float32),
    )(x, feat, cnt3, wa, wb, ba, bb)


def _tc2_kernel(s_ref, u_ref, cnt_ref, wa_ref, wb_ref, ba_ref, bb_ref,
                uo_ref):
    dis = _dis_of(cnt_ref)
    ha = jnp.maximum(dis * (s_ref[0] + u_ref[0]), 0.0)
    hb = jnp.maximum(dis * (s_ref[1] + u_ref[1]), 0.0)
    ta = (jnp.dot(ha, wa_ref[0:H, :], preferred_element_type=jnp.float32)
          + jnp.dot(hb, wa_ref[H:2 * H, :], preferred_element_type=jnp.float32)
          + ba_ref[...])
    tb = (jnp.dot(ha, wb_ref[0:H, :], preferred_element_type=jnp.float32)
          + jnp.dot(hb, wb_ref[H:2 * H, :], preferred_element_type=jnp.float32)
          + bb_ref[...])
    uo_ref[0] = dis * ta
    uo_ref[1] = dis * tb


def _tc2_call(s1, u1, cnt3, wa, wb, ba, bb):
    full = lambda shape: pl.BlockSpec(shape, lambda i: tuple(0 for _ in shape))
    return pl.pallas_call(
        _tc2_kernel,
        grid=(GRID,),
        in_specs=[
            pl.BlockSpec((2, BLK, H), lambda i: (0, i, 0)),
            pl.BlockSpec((2, BLK, H), lambda i: (0, i, 0)),
            pl.BlockSpec((2, BLK, 1), lambda i: (0, i, 0)),
            full((2 * H, H)), full((2 * H, H)), full((1, H)), full((1, H)),
        ],
        out_specs=pl.BlockSpec((2, BLK, H), lambda i: (0, i, 0)),
        out_shape=jax.ShapeDtypeStruct((2, N, H), jnp.float32),
    )(s1, u1, cnt3, wa, wb, ba, bb)


def _tc3_kernel(s_ref, u_ref, cnt_ref, w_ref, b_ref, o_ref):
    dis = _dis_of(cnt_ref)
    ha = jnp.maximum(dis * (s_ref[0] + u_ref[0]), 0.0)
    hb = jnp.maximum(dis * (s_ref[1] + u_ref[1]), 0.0)
    t = (jnp.dot(ha, w_ref[0:H, :], preferred_element_type=jnp.float32)
         + jnp.dot(hb, w_ref[H:2 * H, :], preferred_element_type=jnp.float32)
         + b_ref[...])
    o_ref[...] = jnp.maximum(t, 0.0)


def _tc3_call(s2, u2, cnt3, w, b):
    full = lambda shape: pl.BlockSpec(shape, lambda i: tuple(0 for _ in shape))
    return pl.pallas_call(
        _tc3_kernel,
        grid=(GRID,),
        in_specs=[
            pl.BlockSpec((2, BLK, H), lambda i: (0, i, 0)),
            pl.BlockSpec((2, BLK, H), lambda i: (0, i, 0)),
            pl.BlockSpec((2, BLK, 1), lambda i: (0, i, 0)),
            full((2 * H, 64)), full((1, 64)),
        ],
        out_specs=pl.BlockSpec((BLK, 64), lambda i: (i, 0)),
        out_shape=jax.ShapeDtypeStruct((N, 64), jnp.float32),
    )(s2, u2, cnt3, w, b)


def kernel(x, feat, edge_index, W1, b1, W2, b2, Wfc, bfc):
    row = edge_index[0]
    col = edge_index[1]
    pad = jnp.full((EPAD - E,), N, jnp.int32)
    rowp = jnp.concatenate([row, pad]).reshape(NSUB, 128)
    colp = jnp.concatenate([col, pad]).reshape(NSUB, 128)

    w1t = W1.T
    wa1, wb1 = w1t[:, :H], w1t[:, H:]
    ba1, bb1 = b1[:H].reshape(1, H), b1[H:].reshape(1, H)
    w2t = W2.T
    wa2, wb2 = w2t[:, :H], w2t[:, H:]
    ba2, bb2 = b2[:H].reshape(1, H), b2[H:].reshape(1, H)
    wfct = Wfc.T
    bfc2 = bfc.reshape(1, 64)

    cnt = _deg_call(rowp)
    cnt3 = cnt.reshape(2, ACC_ROWS, 1)
    rows2 = _prep_call(rowp)

    u1 = _tc1_call(x, feat, cnt3, wa1, wb1, ba1, bb1)
    s1 = _prop_call(u1.reshape(2 * N, H), rows2, colp)
    u2 = _tc2_call(s1.reshape(2, N, H), u1, cnt3, wa2, wb2, ba2, bb2)
    s2 = _prop_call(u2.reshape(2 * N, H), rows2, colp)
    return _tc3_call(s2.reshape(2, N, H), u2, cnt3, wfct, bfc2)


# speedup vs baseline: 15.4142x; 1.0176x over previous
import functools

import jax
import jax.numpy as jnp
from jax import lax
from jax.experimental import pallas as pl
from jax.experimental.pallas import tpu as pltpu
from jax.experimental.pallas import tpu_sc as plsc

N = 50000
E = 800000
H = 32
NSUB = 6400
EPAD = NSUB * 128
ACC_ROWS = N + 128
RPT = 3128
BLK = 1000
GRID = N // BLK

_mesh = plsc.VectorSubcoreMesh(core_axis_name="c", subcore_axis_name="s")
_sc_params = pltpu.CompilerParams(use_tc_tiling_on_sc=False)


def _deg_body(row_hbm, cnt_hbm, acc, ibuf, ones, zbuf, dbuf):
    c = lax.axis_index("c")
    s = lax.axis_index("s")
    w = s * 2 + c

    def fill(i, _):
        ones[pl.ds(i * 16, 16)] = jnp.ones((16,), jnp.float32)
        return 0
    lax.fori_loop(0, 8, fill, 0)

    def zfill(i, _):
        zbuf[pl.ds(i * 16, 16)] = jnp.zeros((16,), jnp.float32)
        return 0
    lax.fori_loop(0, 201, zfill, 0)
    pltpu.sync_copy(zbuf.at[pl.ds(0, RPT)], acc.at[pl.ds(s * RPT, RPT)])

    @pl.when(s == 15)
    def _():
        pltpu.sync_copy(zbuf.at[pl.ds(0, 80)], acc.at[pl.ds(16 * RPT, 80)])
    plsc.subcore_barrier()

    def step(g, _):
        sub0 = w * 200 + g * 8
        pltpu.sync_copy(row_hbm.at[pl.ds(sub0, 8)], ibuf)
        for j in range(8):
            pltpu.sync_copy(ones, acc.at[ibuf.at[j]], add=True)
        return 0
    lax.fori_loop(0, 25, step, 0)
    plsc.subcore_barrier()
    pltpu.sync_copy(acc.at[pl.ds(s * RPT, RPT)], dbuf)
    pltpu.sync_copy(dbuf, cnt_hbm.at[pl.ds(c * ACC_ROWS + s * RPT, RPT)])


_deg_call = functools.partial(
    pl.kernel,
    out_type=jax.ShapeDtypeStruct((2 * ACC_ROWS,), jnp.float32),
    mesh=_mesh,
    scratch_types=[
        pltpu.VMEM_SHARED((ACC_ROWS,), jnp.float32),
        pltpu.VMEM((8, 128), jnp.int32),
        pltpu.VMEM((128,), jnp.float32),
        pltpu.VMEM((3216,), jnp.float32),
        pltpu.VMEM((RPT,), jnp.float32),
    ],
    compiler_params=_sc_params,
)(_deg_body)


NSLOT = 4
IB = 8
NB = 400 // IB


def _prop_body(u_hbm, rows2_hbm, col_hbm, s_hbm, acc, ibufr, ibufc, mbuf,
               sem0, sem1, sem2, sem3):
    c = lax.axis_index("c")
    s = lax.axis_index("s")
    sems = (sem0, sem1, sem2, sem3)
    zsrc = mbuf.at[0]

    def zb(i, _):
        mbuf[0, i, 0:16] = jnp.zeros((16,), jnp.float32)
        mbuf[0, i, 16:32] = jnp.zeros((16,), jnp.float32)
        return 0
    lax.fori_loop(0, 128, zb, 0)

    def zc(k, _):
        pltpu.sync_copy(zsrc, acc.at[pl.ds(s * RPT + k * 128, 128)])
        return 0
    lax.fori_loop(0, 24, zc, 0)
    pltpu.sync_copy(zsrc.at[pl.ds(0, 56)],
                    acc.at[pl.ds(s * RPT + 24 * 128, 56)])

    @pl.when(s == 15)
    def _():
        pltpu.sync_copy(zsrc.at[pl.ds(0, 80)], acc.at[pl.ds(16 * RPT, 80)])
    plsc.subcore_barrier()

    def wait_slot(slot):
        pltpu.make_async_copy(u_hbm.at[ibufr.at[0, 0]], mbuf.at[slot],
                              sems[slot]).wait()

    def load_idx(b, par):
        pltpu.sync_copy(
            rows2_hbm.at[pl.ds(c * NSUB + s * 400 + b * IB, IB)],
            ibufr.at[par])
        pltpu.sync_copy(col_hbm.at[pl.ds(s * 400 + b * IB, IB)],
                        ibufc.at[par])

    def gather(slot, par, j):
        pltpu.async_copy(u_hbm.at[ibufr.at[par, j]], mbuf.at[slot],
                         sems[slot])

    def scatter(slot, par, j):
        pltpu.async_copy(mbuf.at[slot], acc.at[ibufc.at[par, j]],
                         sems[slot], add=True)

    def batch_body(b, p):
        for j in range(IB):
            slot = j % NSLOT
            if j < NSLOT:
                @pl.when(b >= 1)
                def _():
                    wait_slot(slot)
            else:
                wait_slot(slot)
            gather(slot, p, j)
            if j == 4:
                @pl.when(b < NB - 1)
                def _():
                    load_idx(b + 1, 1 - p)
            jb = j - 2
            slotb = jb % NSLOT
            if jb < 0:
                @pl.when(b >= 1)
                def _():
                    _seq = (wait_slot(slotb), scatter(slotb, 1 - p, jb + IB))
            else:
                wait_slot(slotb)
                scatter(slotb, p, jb)

    load_idx(jnp.int32(0), 0)

    def outer(bb, _):
        batch_body(2 * bb, 0)
        batch_body(2 * bb + 1, 1)
        return 0
    lax.fori_loop(0, NB // 2, outer, 0)

    for j in (6, 7):
        slot = j % NSLOT
        wait_slot(slot)
        scatter(slot, 1, j)
    for slot in range(NSLOT):
        wait_slot(slot)
    plsc.subcore_barrier()

    bounce = mbuf.at[0]

    def cp(k, _):
        pltpu.sync_copy(acc.at[pl.ds(s * RPT + k * 128, 128)], bounce)
        pltpu.sync_copy(bounce, s_hbm.at[pl.ds(c * N + s * RPT + k * 128, 128)])
        return 0
    lax.fori_loop(0, 24, cp, 0)

    @pl.when(s < 15)
    def _():
        pltpu.sync_copy(acc.at[pl.ds(s * RPT + 3072, 56)],
                        bounce.at[pl.ds(0, 56)])
        pltpu.sync_copy(bounce.at[pl.ds(0, 56)],
                        s_hbm.at[pl.ds(c * N + s * RPT + 3072, 56)])

    @pl.when(s == 15)
    def _():
        pltpu.sync_copy(acc.at[pl.ds(15 * RPT + 3072, 8)],
                        bounce.at[pl.ds(0, 8)])
        pltpu.sync_copy(bounce.at[pl.ds(0, 8)],
                        s_hbm.at[pl.ds(c * N + 15 * RPT + 3072, 8)])


_prop_call = functools.partial(
    pl.kernel,
    out_type=jax.ShapeDtypeStruct((2 * N, H), jnp.float32),
    mesh=_mesh,
    scratch_types=[
        pltpu.VMEM_SHARED((ACC_ROWS, H), jnp.float32),
        pltpu.VMEM((2, IB, 128), jnp.int32),
        pltpu.VMEM((2, IB, 128), jnp.int32),
        pltpu.VMEM((NSLOT, 128, H), jnp.float32),
        pltpu.SemaphoreType.DMA,
        pltpu.SemaphoreType.DMA,
        pltpu.SemaphoreType.DMA,
        pltpu.SemaphoreType.DMA,
    ],
    compiler_params=_sc_params,
)(_prop_body)


def _prep_kernel(row_ref, out_ref):
    c = pl.program_id(0)
    v = row_ref[...]
    out_ref[...] = jnp.where(v >= N, 0, v + c * N)


def _prep_call(rowp):
    return pl.pallas_call(
        _prep_kernel,
        grid=(2, NSUB // 128),
        in_specs=[pl.BlockSpec((128, 128), lambda c, i: (i, 0))],
        out_specs=pl.BlockSpec((128, 128), lambda c, i: (c * (NSUB // 128) + i, 0)),
        out_shape=jax.ShapeDtypeStruct((2 * NSUB // 128 * 128, 128), jnp.int32),
    )(rowp)


def _dis_of(cnt_ref):
    deg = cnt_ref[0, :, 0:1] + cnt_ref[1, :, 0:1] + 1.0
    return lax.rsqrt(deg)


def _tc1_kernel(x_ref, f_ref, cnt_ref, wa_ref, wb_ref, ba_ref, bb_ref, u_ref):
    dis = _dis_of(cnt_ref)
    x = x_ref[...]
    f = f_ref[...]
    ta = (jnp.dot(x, wa_ref[0:64, :], preferred_element_type=jnp.float32)
          + jnp.dot(f, wa_ref[64:128, :], preferred_element_type=jnp.float32)
          + ba_ref[...])
    tb = (jnp.dot(x, wb_ref[0:64, :], preferred_element_type=jnp.float32)
          + jnp.dot(f, wb_ref[64:128, :], preferred_element_type=jnp.float32)
          + bb_ref[...])
    u_ref[0] = dis * ta
    u_ref[1] = dis * tb


def _tc1_call(x, feat, cnt3, wa, wb, ba, bb):
    full = lambda shape: pl.BlockSpec(shape, lambda i: tuple(0 for _ in shape))
    return pl.pallas_call(
        _tc1_kernel,
        grid=(GRID,),
        in_specs=[
            pl.BlockSpec((BLK, 64), lambda i: (i, 0)),
            pl.BlockSpec((BLK, 64), lambda i: (i, 0)),
            pl.BlockSpec((2, BLK, 1), lambda i: (0, i, 0)),
            full((128, H)), full((128, H)), full((1, H)), full((1, H)),
        ],
        out_specs=pl.BlockSpec((2, BLK, H), lambda i: (0, i, 0)),
        out_shape=jax.ShapeDtypeStruct((2, N, H), jnp.float32),
    )(x, feat, cnt3, wa, wb, ba, bb)


def _tc2_kernel(s_ref, u_ref, cnt_ref, wa_ref, wb_ref, ba_ref, bb_ref,
                uo_ref):
    dis = _dis_of(cnt_ref)
    ha = jnp.maximum(dis * (s_ref[0] + u_ref[0]), 0.0)
    hb = jnp.maximum(dis * (s_ref[1] + u_ref[1]), 0.0)
    ta = (jnp.dot(ha, wa_ref[0:H, :], preferred_element_type=jnp.float32)
          + jnp.dot(hb, wa_ref[H:2 * H, :], preferred_element_type=jnp.float32)
          + ba_ref[...])
    tb = (jnp.dot(ha, wb_ref[0:H, :], preferred_element_type=jnp.float32)
          + jnp.dot(hb, wb_ref[H:2 * H, :], preferred_element_type=jnp.float32)
          + bb_ref[...])
    uo_ref[0] = dis * ta
    uo_ref[1] = dis * tb


def _tc2_call(s1, u1, cnt3, wa, wb, ba, bb):
    full = lambda shape: pl.BlockSpec(shape, lambda i: tuple(0 for _ in shape))
    return pl.pallas_call(
        _tc2_kernel,
        grid=(GRID,),
        in_specs=[
            pl.BlockSpec((2, BLK, H), lambda i: (0, i, 0)),
            pl.BlockSpec((2, BLK, H), lambda i: (0, i, 0)),
            pl.BlockSpec((2, BLK, 1), lambda i: (0, i, 0)),
            full((2 * H, H)), full((2 * H, H)), full((1, H)), full((1, H)),
        ],
        out_specs=pl.BlockSpec((2, BLK, H), lambda i: (0, i, 0)),
        out_shape=jax.ShapeDtypeStruct((2, N, H), jnp.float32),
    )(s1, u1, cnt3, wa, wb, ba, bb)


def _tc3_kernel(s_ref, u_ref, cnt_ref, w_ref, b_ref, o_ref):
    dis = _dis_of(cnt_ref)
    ha = jnp.maximum(dis * (s_ref[0] + u_ref[0]), 0.0)
    hb = jnp.maximum(dis * (s_ref[1] + u_ref[1]), 0.0)
    t = (jnp.dot(ha, w_ref[0:H, :], preferred_element_type=jnp.float32)
         + jnp.dot(hb, w_ref[H:2 * H, :], preferred_element_type=jnp.float32)
         + b_ref[...])
    o_ref[...] = jnp.maximum(t, 0.0)


def _tc3_call(s2, u2, cnt3, w, b):
    full = lambda shape: pl.BlockSpec(shape, lambda i: tuple(0 for _ in shape))
    return pl.pallas_call(
        _tc3_kernel,
        grid=(GRID,),
        in_specs=[
            pl.BlockSpec((2, BLK, H), lambda i: (0, i, 0)),
            pl.BlockSpec((2, BLK, H), lambda i: (0, i, 0)),
            pl.BlockSpec((2, BLK, 1), lambda i: (0, i, 0)),
            full((2 * H, 64)), full((1, 64)),
        ],
        out_specs=pl.BlockSpec((BLK, 64), lambda i: (i, 0)),
        out_shape=jax.ShapeDtypeStruct((N, 64), jnp.float32),
    )(s2, u2, cnt3, w, b)


def kernel(x, feat, edge_index, W1, b1, W2, b2, Wfc, bfc):
    row = edge_index[0]
    col = edge_index[1]
    pad = jnp.full((EPAD - E,), N, jnp.int32)
    rowp = jnp.concatenate([row, pad]).reshape(NSUB, 128)
    colp = jnp.concatenate([col, pad]).reshape(NSUB, 128)

    w1t = W1.T
    wa1, wb1 = w1t[:, :H], w1t[:, H:]
    ba1, bb1 = b1[:H].reshape(1, H), b1[H:].reshape(1, H)
    w2t = W2.T
    wa2, wb2 = w2t[:, :H], w2t[:, H:]
    ba2, bb2 = b2[:H].reshape(1, H), b2[H:].reshape(1, H)
    wfct = Wfc.T
    bfc2 = bfc.reshape(1, 64)

    cnt = _deg_call(rowp)
    cnt3 = cnt.reshape(2, ACC_ROWS, 1)
    rows2 = _prep_call(rowp)

    u1 = _tc1_call(x, feat, cnt3, wa1, wb1, ba1, bb1)
    s1 = _prop_call(u1.reshape(2 * N, H), rows2, colp)
    u2 = _tc2_call(s1.reshape(2, N, H), u1, cnt3, wa2, wb2, ba2, bb2)
    s2 = _prop_call(u2.reshape(2 * N, H), rows2, colp)
    return _tc3_call(s2.reshape(2, N, H), u2, cnt3, wfct, bfc2)


# speedup vs baseline: 16.2188x; 1.0522x over previous
import functools

import jax
import jax.numpy as jnp
from jax import lax
from jax.experimental import pallas as pl
from jax.experimental.pallas import tpu as pltpu
from jax.experimental.pallas import tpu_sc as plsc

N = 50000
E = 800000
H = 32
NSUB = 6400
EPAD = NSUB * 128
ACC_ROWS = N + 128
RPT = 3128
ACC_D = N + 176
RPT_D = ACC_D // 16
BLK = 1024
GRID = 49

_mesh = plsc.VectorSubcoreMesh(core_axis_name="c", subcore_axis_name="s")
_sc_params = pltpu.CompilerParams(use_tc_tiling_on_sc=False)


def _deg_body(row_hbm, cnt_hbm, acc, ibuf, ones, zbuf, dbuf):
    c = lax.axis_index("c")
    s = lax.axis_index("s")
    w = s * 2 + c

    def fill(i, _):
        ones[pl.ds(i * 16, 16)] = jnp.ones((16,), jnp.float32)
        return 0
    lax.fori_loop(0, 8, fill, 0)

    def zfill(i, _):
        zbuf[pl.ds(i * 16, 16)] = jnp.zeros((16,), jnp.float32)
        return 0
    lax.fori_loop(0, RPT_D // 16, zfill, 0)
    pltpu.sync_copy(zbuf, acc.at[pl.ds(s * RPT_D, RPT_D)])
    plsc.subcore_barrier()

    def step(g, _):
        sub0 = w * 200 + g * 8
        pltpu.sync_copy(row_hbm.at[pl.ds(sub0, 8)], ibuf)
        for j in range(8):
            pltpu.sync_copy(ones, acc.at[ibuf.at[j]], add=True)
        return 0
    lax.fori_loop(0, 25, step, 0)
    plsc.subcore_barrier()
    pltpu.sync_copy(acc.at[pl.ds(s * RPT_D, RPT_D)], dbuf)
    pltpu.sync_copy(dbuf, cnt_hbm.at[pl.ds(c * ACC_D + s * RPT_D, RPT_D)])


_deg_call = functools.partial(
    pl.kernel,
    out_type=jax.ShapeDtypeStruct((2 * ACC_D,), jnp.float32),
    mesh=_mesh,
    scratch_types=[
        pltpu.VMEM_SHARED((ACC_D,), jnp.float32),
        pltpu.VMEM((8, 128), jnp.int32),
        pltpu.VMEM((128,), jnp.float32),
        pltpu.VMEM((RPT_D,), jnp.float32),
        pltpu.VMEM((RPT_D,), jnp.float32),
    ],
    compiler_params=_sc_params,
)(_deg_body)


NSLOT = 4
IB = 8
NB = 400 // IB


def _prop_body(u_hbm, rows2_hbm, col_hbm, s_hbm, acc, ibufr, ibufc, mbuf,
               sem0, sem1, sem2, sem3):
    c = lax.axis_index("c")
    s = lax.axis_index("s")
    sems = (sem0, sem1, sem2, sem3)
    zsrc = mbuf.at[0]

    def zb(i, _):
        mbuf[0, i, 0:16] = jnp.zeros((16,), jnp.float32)
        mbuf[0, i, 16:32] = jnp.zeros((16,), jnp.float32)
        return 0
    lax.fori_loop(0, 128, zb, 0)

    def zc(k, _):
        pltpu.sync_copy(zsrc, acc.at[pl.ds(s * RPT + k * 128, 128)])
        return 0
    lax.fori_loop(0, 24, zc, 0)
    pltpu.sync_copy(zsrc.at[pl.ds(0, 56)],
                    acc.at[pl.ds(s * RPT + 24 * 128, 56)])

    @pl.when(s == 15)
    def _():
        pltpu.sync_copy(zsrc.at[pl.ds(0, 80)], acc.at[pl.ds(16 * RPT, 80)])
    plsc.subcore_barrier()

    def wait_slot(slot):
        pltpu.make_async_copy(u_hbm.at[ibufr.at[0, 0]], mbuf.at[slot],
                              sems[slot]).wait()

    def load_idx(b, par):
        pltpu.sync_copy(
            rows2_hbm.at[pl.ds(c * NSUB + s * 400 + b * IB, IB)],
            ibufr.at[par])
        pltpu.sync_copy(col_hbm.at[pl.ds(s * 400 + b * IB, IB)],
                        ibufc.at[par])

    def gather(slot, par, j):
        pltpu.async_copy(u_hbm.at[ibufr.at[par, j]], mbuf.at[slot],
                         sems[slot])

    def scatter(slot, par, j):
        pltpu.async_copy(mbuf.at[slot], acc.at[ibufc.at[par, j]],
                         sems[slot], add=True)

    def batch_body(b, p):
        for j in range(IB):
            slot = j % NSLOT
            if j < NSLOT:
                @pl.when(b >= 1)
                def _():
                    wait_slot(slot)
            else:
                wait_slot(slot)
            gather(slot, p, j)
            if j == 4:
                @pl.when(b < NB - 1)
                def _():
                    load_idx(b + 1, 1 - p)
            jb = j - 2
            slotb = jb % NSLOT
            if jb < 0:
                @pl.when(b >= 1)
                def _():
                    _seq = (wait_slot(slotb), scatter(slotb, 1 - p, jb + IB))
            else:
                wait_slot(slotb)
                scatter(slotb, p, jb)

    load_idx(jnp.int32(0), 0)

    def outer(bb, _):
        batch_body(2 * bb, 0)
        batch_body(2 * bb + 1, 1)
        return 0
    lax.fori_loop(0, NB // 2, outer, 0)

    for j in (6, 7):
        slot = j % NSLOT
        wait_slot(slot)
        scatter(slot, 1, j)
    for slot in range(NSLOT):
        wait_slot(slot)
    plsc.subcore_barrier()

    bounce = mbuf.at[0]

    def cp(k, _):
        pltpu.sync_copy(acc.at[pl.ds(s * RPT + k * 128, 128)], bounce)
        pltpu.sync_copy(bounce, s_hbm.at[pl.ds(c * N + s * RPT + k * 128, 128)])
        return 0
    lax.fori_loop(0, 24, cp, 0)

    @pl.when(s < 15)
    def _():
        pltpu.sync_copy(acc.at[pl.ds(s * RPT + 3072, 56)],
                        bounce.at[pl.ds(0, 56)])
        pltpu.sync_copy(bounce.at[pl.ds(0, 56)],
                        s_hbm.at[pl.ds(c * N + s * RPT + 3072, 56)])

    @pl.when(s == 15)
    def _():
        pltpu.sync_copy(acc.at[pl.ds(15 * RPT + 3072, 8)],
                        bounce.at[pl.ds(0, 8)])
        pltpu.sync_copy(bounce.at[pl.ds(0, 8)],
                        s_hbm.at[pl.ds(c * N + 15 * RPT + 3072, 8)])


_prop_call = functools.partial(
    pl.kernel,
    out_type=jax.ShapeDtypeStruct((2 * N, H), jnp.float32),
    mesh=_mesh,
    scratch_types=[
        pltpu.VMEM_SHARED((ACC_ROWS, H), jnp.float32),
        pltpu.VMEM((2, IB, 128), jnp.int32),
        pltpu.VMEM((2, IB, 128), jnp.int32),
        pltpu.VMEM((NSLOT, 128, H), jnp.float32),
        pltpu.SemaphoreType.DMA,
        pltpu.SemaphoreType.DMA,
        pltpu.SemaphoreType.DMA,
        pltpu.SemaphoreType.DMA,
    ],
    compiler_params=_sc_params,
)(_prop_body)


def _prep_kernel(row_ref, out_ref):
    c = pl.program_id(0)
    v = row_ref[...]
    out_ref[...] = jnp.where(v >= N, 0, v + c * N)


def _prep_call(rowp):
    return pl.pallas_call(
        _prep_kernel,
        grid=(2, NSUB // 128),
        in_specs=[pl.BlockSpec((128, 128), lambda c, i: (i, 0))],
        out_specs=pl.BlockSpec((128, 128), lambda c, i: (c * (NSUB // 128) + i, 0)),
        out_shape=jax.ShapeDtypeStruct((2 * NSUB // 128 * 128, 128), jnp.int32),
    )(rowp)


def _dis_of(ca_ref, cb_ref):
    deg = ca_ref[0] + cb_ref[0] + 1.0
    return lax.transpose(lax.rsqrt(deg), (1, 0))


def _tc1_kernel(x_ref, f_ref, ca_ref, cb_ref, wa_ref, wb_ref, ba_ref, bb_ref,
                u_ref):
    dis = _dis_of(ca_ref, cb_ref)
    x = x_ref[...]
    f = f_ref[...]
    ta = (jnp.dot(x, wa_ref[0:64, :], preferred_element_type=jnp.float32)
          + jnp.dot(f, wa_ref[64:128, :], preferred_element_type=jnp.float32)
          + ba_ref[...])
    tb = (jnp.dot(x, wb_ref[0:64, :], preferred_element_type=jnp.float32)
          + jnp.dot(f, wb_ref[64:128, :], preferred_element_type=jnp.float32)
          + bb_ref[...])
    u_ref[0] = dis * ta
    u_ref[1] = dis * tb


_CNTA = pl.BlockSpec((1, 1, BLK), lambda i: (i, 0, 0))
_CNTB = pl.BlockSpec((1, 1, BLK), lambda i: (ACC_D // BLK + i, 0, 0))


def _tc1_call(x, feat, cnt4, wa, wb, ba, bb):
    full = lambda shape: pl.BlockSpec(shape, lambda i: tuple(0 for _ in shape))
    return pl.pallas_call(
        _tc1_kernel,
        grid=(GRID,),
        in_specs=[
            pl.BlockSpec((BLK, 64), lambda i: (i, 0)),
            pl.BlockSpec((BLK, 64), lambda i: (i, 0)),
            _CNTA, _CNTB,
            full((128, H)), full((128, H)), full((1, H)), full((1, H)),
        ],
        out_specs=pl.BlockSpec((2, BLK, H), lambda i: (0, i, 0)),
        out_shape=jax.ShapeDtypeStruct((2, N, H), jnp.float32),
    )(x, feat, cnt4, cnt4, wa, wb, ba, bb)


def _tc2_kernel(s_ref, u_ref, ca_ref, cb_ref, wa_ref, wb_ref, ba_ref, bb_ref,
                uo_ref):
    dis = _dis_of(ca_ref, cb_ref)
    ha = jnp.maximum(dis * (s_ref[0] + u_ref[0]), 0.0)
    hb = jnp.maximum(dis * (s_ref[1] + u_ref[1]), 0.0)
    ta = (jnp.dot(ha, wa_ref[0:H, :], preferred_element_type=jnp.float32)
          + jnp.dot(hb, wa_ref[H:2 * H, :], preferred_element_type=jnp.float32)
          + ba_ref[...])
    tb = (jnp.dot(ha, wb_ref[0:H, :], preferred_element_type=jnp.float32)
          + jnp.dot(hb, wb_ref[H:2 * H, :], preferred_element_type=jnp.float32)
          + bb_ref[...])
    uo_ref[0] = dis * ta
    uo_ref[1] = dis * tb


def _tc2_call(s1, u1, cnt4, wa, wb, ba, bb):
    full = lambda shape: pl.BlockSpec(shape, lambda i: tuple(0 for _ in shape))
    return pl.pallas_call(
        _tc2_kernel,
        grid=(GRID,),
        in_specs=[
            pl.BlockSpec((2, BLK, H), lambda i: (0, i, 0)),
            pl.BlockSpec((2, BLK, H), lambda i: (0, i, 0)),
            _CNTA, _CNTB,
            full((2 * H, H)), full((2 * H, H)), full((1, H)), full((1, H)),
        ],
        out_specs=pl.BlockSpec((2, BLK, H), lambda i: (0, i, 0)),
        out_shape=jax.ShapeDtypeStruct((2, N, H), jnp.float32),
    )(s1, u1, cnt4, cnt4, wa, wb, ba, bb)


def _tc3_kernel(s_ref, u_ref, ca_ref, cb_ref, w_ref, b_ref, o_ref):
    dis = _dis_of(ca_ref, cb_ref)
    ha = jnp.maximum(dis * (s_ref[0] + u_ref[0]), 0.0)
    hb = jnp.maximum(dis * (s_ref[1] + u_ref[1]), 0.0)
    t = (jnp.dot(ha, w_ref[0:H, :], preferred_element_type=jnp.float32)
         + jnp.dot(hb, w_ref[H:2 * H, :], preferred_element_type=jnp.float32)
         + b_ref[...])
    o_ref[...] = jnp.maximum(t, 0.0)


def _tc3_call(s2, u2, cnt4, w, b):
    full = lambda shape: pl.BlockSpec(shape, lambda i: tuple(0 for _ in shape))
    return pl.pallas_call(
        _tc3_kernel,
        grid=(GRID,),
        in_specs=[
            pl.BlockSpec((2, BLK, H), lambda i: (0, i, 0)),
            pl.BlockSpec((2, BLK, H), lambda i: (0, i, 0)),
            _CNTA, _CNTB,
            full((2 * H, 64)), full((1, 64)),
        ],
        out_specs=pl.BlockSpec((BLK, 64), lambda i: (i, 0)),
        out_shape=jax.ShapeDtypeStruct((N, 64), jnp.float32),
    )(s2, u2, cnt4, cnt4, w, b)


def kernel(x, feat, edge_index, W1, b1, W2, b2, Wfc, bfc):
    row = edge_index[0]
    col = edge_index[1]
    pad = jnp.full((EPAD - E,), N, jnp.int32)
    rowp = jnp.concatenate([row, pad]).reshape(NSUB, 128)
    colp = jnp.concatenate([col, pad]).reshape(NSUB, 128)

    w1t = W1.T
    wa1, wb1 = w1t[:, :H], w1t[:, H:]
    ba1, bb1 = b1[:H].reshape(1, H), b1[H:].reshape(1, H)
    w2t = W2.T
    wa2, wb2 = w2t[:, :H], w2t[:, H:]
    ba2, bb2 = b2[:H].reshape(1, H), b2[H:].reshape(1, H)
    wfct = Wfc.T
    bfc2 = bfc.reshape(1, 64)

    cnt = _deg_call(rowp)
    cnt4 = cnt.reshape(2 * ACC_D // BLK, 1, BLK)
    rows2 = _prep_call(rowp)

    u1 = _tc1_call(x, feat, cnt4, wa1, wb1, ba1, bb1)
    s1 = _prop_call(u1.reshape(2 * N, H), rows2, colp)
    u2 = _tc2_call(s1.reshape(2, N, H), u1, cnt4, wa2, wb2, ba2, bb2)
    s2 = _prop_call(u2.reshape(2 * N, H), rows2, colp)
    return _tc3_call(s2.reshape(2, N, H), u2, cnt4, wfct, bfc2)


# speedup vs baseline: 17.0147x; 1.0491x over previous
import functools

import jax
import jax.numpy as jnp
from jax import lax
from jax.experimental import pallas as pl
from jax.experimental.pallas import tpu as pltpu
from jax.experimental.pallas import tpu_sc as plsc

N = 50000
E = 800000
H = 32
NSUB = 6400
EPAD = NSUB * 128
ACC_ROWS = N + 128
RPT = 3128
ACC_D = N + 176
RPT_D = ACC_D // 16
BLK = 1024
GRID = 49

_mesh = plsc.VectorSubcoreMesh(core_axis_name="c", subcore_axis_name="s")
_sc_params = pltpu.CompilerParams(use_tc_tiling_on_sc=False)


def _deg_body(row_hbm, cnt_hbm, acc, ibuf, ones, zbuf, dbuf):
    c = lax.axis_index("c")
    s = lax.axis_index("s")
    w = s * 2 + c

    def fill(i, _):
        ones[pl.ds(i * 16, 16)] = jnp.ones((16,), jnp.float32)
        return 0
    lax.fori_loop(0, 8, fill, 0)

    def zfill(i, _):
        zbuf[pl.ds(i * 16, 16)] = jnp.zeros((16,), jnp.float32)
        return 0
    lax.fori_loop(0, RPT_D // 16, zfill, 0)
    pltpu.sync_copy(zbuf, acc.at[pl.ds(s * RPT_D, RPT_D)])
    plsc.subcore_barrier()

    def step(g, _):
        sub0 = w * 200 + g * 8
        pltpu.sync_copy(row_hbm.at[pl.ds(sub0, 8)], ibuf)
        for j in range(8):
            pltpu.sync_copy(ones, acc.at[ibuf.at[j]], add=True)
        return 0
    lax.fori_loop(0, 25, step, 0)
    plsc.subcore_barrier()
    pltpu.sync_copy(acc.at[pl.ds(s * RPT_D, RPT_D)], dbuf)
    pltpu.sync_copy(dbuf, cnt_hbm.at[pl.ds(c * ACC_D + s * RPT_D, RPT_D)])


_deg_call = functools.partial(
    pl.kernel,
    out_type=jax.ShapeDtypeStruct((2 * ACC_D,), jnp.float32),
    mesh=_mesh,
    scratch_types=[
        pltpu.VMEM_SHARED((ACC_D,), jnp.float32),
        pltpu.VMEM((8, 128), jnp.int32),
        pltpu.VMEM((128,), jnp.float32),
        pltpu.VMEM((RPT_D,), jnp.float32),
        pltpu.VMEM((RPT_D,), jnp.float32),
    ],
    compiler_params=_sc_params,
)(_deg_body)


NSLOT = 5
IB = 10
NB = 400 // IB
LAG = 3


def _prop_body(u_hbm, rows2_hbm, col_hbm, s_hbm, acc, ibufr, ibufc, mbuf,
               sem0, sem1, sem2, sem3, sem4):
    c = lax.axis_index("c")
    s = lax.axis_index("s")
    sems = (sem0, sem1, sem2, sem3, sem4)
    zsrc = mbuf.at[0]

    def zb(i, _):
        mbuf[0, i, 0:16] = jnp.zeros((16,), jnp.float32)
        mbuf[0, i, 16:32] = jnp.zeros((16,), jnp.float32)
        return 0
    lax.fori_loop(0, 128, zb, 0)

    def zc(k, _):
        pltpu.sync_copy(zsrc, acc.at[pl.ds(s * RPT + k * 128, 128)])
        return 0
    lax.fori_loop(0, 24, zc, 0)
    pltpu.sync_copy(zsrc.at[pl.ds(0, 56)],
                    acc.at[pl.ds(s * RPT + 24 * 128, 56)])

    @pl.when(s == 15)
    def _():
        pltpu.sync_copy(zsrc.at[pl.ds(0, 80)], acc.at[pl.ds(16 * RPT, 80)])
    plsc.subcore_barrier()

    def wait_slot(slot):
        pltpu.make_async_copy(u_hbm.at[ibufr.at[0, 0]], mbuf.at[slot],
                              sems[slot]).wait()

    def load_idx(b, par):
        pltpu.sync_copy(
            rows2_hbm.at[pl.ds(c * NSUB + s * 400 + b * IB, IB)],
            ibufr.at[par])
        pltpu.sync_copy(col_hbm.at[pl.ds(s * 400 + b * IB, IB)],
                        ibufc.at[par])

    def gather(slot, par, j):
        pltpu.async_copy(u_hbm.at[ibufr.at[par, j]], mbuf.at[slot],
                         sems[slot])

    def scatter(slot, par, j):
        pltpu.async_copy(mbuf.at[slot], acc.at[ibufc.at[par, j]],
                         sems[slot], add=True)

    def batch_body(b, p):
        for j in range(IB):
            slot = j % NSLOT
            if j < NSLOT:
                @pl.when(b >= 1)
                def _():
                    wait_slot(slot)
            else:
                wait_slot(slot)
            gather(slot, p, j)
            if j == NSLOT:
                @pl.when(b < NB - 1)
                def _():
                    load_idx(b + 1, 1 - p)
            jb = j - LAG
            slotb = jb % NSLOT
            if jb < 0:
                @pl.when(b >= 1)
                def _():
                    _seq = (wait_slot(slotb), scatter(slotb, 1 - p, jb + IB))
            else:
                wait_slot(slotb)
                scatter(slotb, p, jb)

    load_idx(jnp.int32(0), 0)

    def outer(bb, _):
        batch_body(2 * bb, 0)
        batch_body(2 * bb + 1, 1)
        return 0
    lax.fori_loop(0, NB // 2, outer, 0)

    for j in range(IB - LAG, IB):
        slot = j % NSLOT
        wait_slot(slot)
        scatter(slot, 1, j)
    for slot in range(NSLOT):
        wait_slot(slot)
    plsc.subcore_barrier()

    bounce = mbuf.at[0]

    def cp(k, _):
        pltpu.sync_copy(acc.at[pl.ds(s * RPT + k * 128, 128)], bounce)
        pltpu.sync_copy(bounce, s_hbm.at[pl.ds(c * N + s * RPT + k * 128, 128)])
        return 0
    lax.fori_loop(0, 24, cp, 0)

    @pl.when(s < 15)
    def _():
        pltpu.sync_copy(acc.at[pl.ds(s * RPT + 3072, 56)],
                        bounce.at[pl.ds(0, 56)])
        pltpu.sync_copy(bounce.at[pl.ds(0, 56)],
                        s_hbm.at[pl.ds(c * N + s * RPT + 3072, 56)])

    @pl.when(s == 15)
    def _():
        pltpu.sync_copy(acc.at[pl.ds(15 * RPT + 3072, 8)],
                        bounce.at[pl.ds(0, 8)])
        pltpu.sync_copy(bounce.at[pl.ds(0, 8)],
                        s_hbm.at[pl.ds(c * N + 15 * RPT + 3072, 8)])


_prop_call = functools.partial(
    pl.kernel,
    out_type=jax.ShapeDtypeStruct((2 * N, H), jnp.float32),
    mesh=_mesh,
    scratch_types=[
        pltpu.VMEM_SHARED((ACC_ROWS, H), jnp.float32),
        pltpu.VMEM((2, IB, 128), jnp.int32),
        pltpu.VMEM((2, IB, 128), jnp.int32),
        pltpu.VMEM((NSLOT, 128, H), jnp.float32),
        pltpu.SemaphoreType.DMA,
        pltpu.SemaphoreType.DMA,
        pltpu.SemaphoreType.DMA,
        pltpu.SemaphoreType.DMA,
        pltpu.SemaphoreType.DMA,
    ],
    compiler_params=_sc_params,
)(_prop_body)


def _prep_kernel(row_ref, out_ref):
    c = pl.program_id(0)
    v = row_ref[...]
    out_ref[...] = jnp.where(v >= N, 0, v + c * N)


def _prep_call(rowp):
    return pl.pallas_call(
        _prep_kernel,
        grid=(2, NSUB // 128),
        in_specs=[pl.BlockSpec((128, 128), lambda c, i: (i, 0))],
        out_specs=pl.BlockSpec((128, 128), lambda c, i: (c * (NSUB // 128) + i, 0)),
        out_shape=jax.ShapeDtypeStruct((2 * NSUB // 128 * 128, 128), jnp.int32),
    )(rowp)


def _dis_of(ca_ref, cb_ref):
    deg = ca_ref[0] + cb_ref[0] + 1.0
    return lax.transpose(lax.rsqrt(deg), (1, 0))


_DN_T = (((0,), (0,)), ((), ()))


def _tc1_kernel(x_ref, f_ref, ca_ref, cb_ref, wa_ref, wb_ref, ba_ref, bb_ref,
                u_ref):
    dis = _dis_of(ca_ref, cb_ref)
    x = x_ref[...]
    f = f_ref[...]
    dot = lambda a, b: lax.dot_general(a, b, _DN_T,
                                       preferred_element_type=jnp.float32)
    ta = dot(x, wa_ref[0:64, :]) + dot(f, wa_ref[64:128, :]) + ba_ref[...]
    tb = dot(x, wb_ref[0:64, :]) + dot(f, wb_ref[64:128, :]) + bb_ref[...]
    u_ref[0] = dis * ta
    u_ref[1] = dis * tb


_CNTA = pl.BlockSpec((1, 1, BLK), lambda i: (i, 0, 0))
_CNTB = pl.BlockSpec((1, 1, BLK), lambda i: (ACC_D // BLK + i, 0, 0))


def _tc1_call(x, feat, cnt4, wa, wb, ba, bb):
    full = lambda shape: pl.BlockSpec(shape, lambda i: tuple(0 for _ in shape))
    return pl.pallas_call(
        _tc1_kernel,
        grid=(GRID,),
        in_specs=[
            pl.BlockSpec((64, BLK), lambda i: (0, i)),
            pl.BlockSpec((64, BLK), lambda i: (0, i)),
            _CNTA, _CNTB,
            full((128, H)), full((128, H)), full((1, H)), full((1, H)),
        ],
        out_specs=pl.BlockSpec((2, BLK, H), lambda i: (0, i, 0)),
        out_shape=jax.ShapeDtypeStruct((2, N, H), jnp.float32),
    )(x, feat, cnt4, cnt4, wa, wb, ba, bb)


def _tc2_kernel(s_ref, u_ref, ca_ref, cb_ref, wa_ref, wb_ref, ba_ref, bb_ref,
                uo_ref):
    dis = _dis_of(ca_ref, cb_ref)
    ha = jnp.maximum(dis * (s_ref[0] + u_ref[0]), 0.0)
    hb = jnp.maximum(dis * (s_ref[1] + u_ref[1]), 0.0)
    ta = (jnp.dot(ha, wa_ref[0:H, :], preferred_element_type=jnp.float32)
          + jnp.dot(hb, wa_ref[H:2 * H, :], preferred_element_type=jnp.float32)
          + ba_ref[...])
    tb = (jnp.dot(ha, wb_ref[0:H, :], preferred_element_type=jnp.float32)
          + jnp.dot(hb, wb_ref[H:2 * H, :], preferred_element_type=jnp.float32)
          + bb_ref[...])
    uo_ref[0] = dis * ta
    uo_ref[1] = dis * tb


def _tc2_call(s1, u1, cnt4, wa, wb, ba, bb):
    full = lambda shape: pl.BlockSpec(shape, lambda i: tuple(0 for _ in shape))
    return pl.pallas_call(
        _tc2_kernel,
        grid=(GRID,),
        in_specs=[
            pl.BlockSpec((2, BLK, H), lambda i: (0, i, 0)),
            pl.BlockSpec((2, BLK, H), lambda i: (0, i, 0)),
            _CNTA, _CNTB,
            full((2 * H, H)), full((2 * H, H)), full((1, H)), full((1, H)),
        ],
        out_specs=pl.BlockSpec((2, BLK, H), lambda i: (0, i, 0)),
        out_shape=jax.ShapeDtypeStruct((2, N, H), jnp.float32),
    )(s1, u1, cnt4, cnt4, wa, wb, ba, bb)


def _tc3_kernel(s_ref, u_ref, ca_ref, cb_ref, w_ref, b_ref, o_ref):
    dis = _dis_of(ca_ref, cb_ref)
    ha = jnp.maximum(dis * (s_ref[0] + u_ref[0]), 0.0)
    hb = jnp.maximum(dis * (s_ref[1] + u_ref[1]), 0.0)
    t = (jnp.dot(ha, w_ref[0:H, :], preferred_element_type=jnp.float32)
         + jnp.dot(hb, w_ref[H:2 * H, :], preferred_element_type=jnp.float32)
         + b_ref[...])
    o_ref[...] = jnp.maximum(t, 0.0)


def _tc3_call(s2, u2, cnt4, w, b):
    full = lambda shape: pl.BlockSpec(shape, lambda i: tuple(0 for _ in shape))
    return pl.pallas_call(
        _tc3_kernel,
        grid=(GRID,),
        in_specs=[
            pl.BlockSpec((2, BLK, H), lambda i: (0, i, 0)),
            pl.BlockSpec((2, BLK, H), lambda i: (0, i, 0)),
            _CNTA, _CNTB,
            full((2 * H, 64)), full((1, 64)),
        ],
        out_specs=pl.BlockSpec((BLK, 64), lambda i: (i, 0)),
        out_shape=jax.ShapeDtypeStruct((N, 64), jnp.float32),
    )(s2, u2, cnt4, cnt4, w, b)


def kernel(x, feat, edge_index, W1, b1, W2, b2, Wfc, bfc):
    row = edge_index[0]
    col = edge_index[1]
    pad = jnp.full((EPAD - E,), N, jnp.int32)
    rowp = jnp.concatenate([row, pad]).reshape(NSUB, 128)
    colp = jnp.concatenate([col, pad]).reshape(NSUB, 128)

    w1t = W1.T
    wa1, wb1 = w1t[:, :H], w1t[:, H:]
    ba1, bb1 = b1[:H].reshape(1, H), b1[H:].reshape(1, H)
    w2t = W2.T
    wa2, wb2 = w2t[:, :H], w2t[:, H:]
    ba2, bb2 = b2[:H].reshape(1, H), b2[H:].reshape(1, H)
    wfct = Wfc.T
    bfc2 = bfc.reshape(1, 64)

    cnt = _deg_call(rowp)
    cnt4 = cnt.reshape(2 * ACC_D // BLK, 1, BLK)
    rows2 = _prep_call(rowp)

    u1 = _tc1_call(x.T, feat.T, cnt4, wa1, wb1, ba1, bb1)
    s1 = _prop_call(u1.reshape(2 * N, H), rows2, colp)
    u2 = _tc2_call(s1.reshape(2, N, H), u1, cnt4, wa2, wb2, ba2, bb2)
    s2 = _prop_call(u2.reshape(2 * N, H), rows2, colp)
    return _tc3_call(s2.reshape(2, N, H), u2, cnt4, wfct, bfc2)


# speedup vs baseline: 17.2069x; 1.0113x over previous
import functools

import jax
import jax.numpy as jnp
from jax import lax
from jax.experimental import pallas as pl
from jax.experimental.pallas import tpu as pltpu
from jax.experimental.pallas import tpu_sc as plsc

N = 50000
E = 800000
H = 32
NSUB = 6400
EPAD = NSUB * 128
ACC_ROWS = N + 128
RPT = 3128
ACC_D = N + 176
RPT_D = ACC_D // 16
BLK = 1024
GRID = 49

_mesh = plsc.VectorSubcoreMesh(core_axis_name="c", subcore_axis_name="s")
_sc_params = pltpu.CompilerParams(use_tc_tiling_on_sc=False)


def _deg_body(row_hbm, cnt_hbm, acc, ibuf, ones, zbuf, dbuf):
    c = lax.axis_index("c")
    s = lax.axis_index("s")
    w = s * 2 + c

    def fill(i, _):
        ones[pl.ds(i * 16, 16)] = jnp.ones((16,), jnp.float32)
        return 0
    lax.fori_loop(0, 8, fill, 0)

    def zfill(i, _):
        zbuf[pl.ds(i * 16, 16)] = jnp.zeros((16,), jnp.float32)
        return 0
    lax.fori_loop(0, RPT_D // 16, zfill, 0)
    pltpu.sync_copy(zbuf, acc.at[pl.ds(s * RPT_D, RPT_D)])
    plsc.subcore_barrier()

    def step(g, _):
        sub0 = w * 200 + g * 8
        pltpu.sync_copy(row_hbm.at[pl.ds(sub0, 8)], ibuf)
        for j in range(8):
            pltpu.sync_copy(ones, acc.at[ibuf.at[j]], add=True)
        return 0
    lax.fori_loop(0, 25, step, 0)
    plsc.subcore_barrier()
    pltpu.sync_copy(acc.at[pl.ds(s * RPT_D, RPT_D)], dbuf)
    pltpu.sync_copy(dbuf, cnt_hbm.at[pl.ds(c * ACC_D + s * RPT_D, RPT_D)])


_deg_call = functools.partial(
    pl.kernel,
    out_type=jax.ShapeDtypeStruct((2 * ACC_D,), jnp.float32),
    mesh=_mesh,
    scratch_types=[
        pltpu.VMEM_SHARED((ACC_D,), jnp.float32),
        pltpu.VMEM((8, 128), jnp.int32),
        pltpu.VMEM((128,), jnp.float32),
        pltpu.VMEM((RPT_D,), jnp.float32),
        pltpu.VMEM((RPT_D,), jnp.float32),
    ],
    compiler_params=_sc_params,
)(_deg_body)


NSLOT = 5
IB = 10
NB = 400 // IB
LAG = 3


def _prop_body(u_hbm, rows2_hbm, col_hbm, s_hbm, acc, ibufr, ibufc, mbuf,
               sem0, sem1, sem2, sem3, sem4):
    c = lax.axis_index("c")
    s = lax.axis_index("s")
    sems = (sem0, sem1, sem2, sem3, sem4)
    zsrc = mbuf.at[0]

    def zb(i, _):
        mbuf[0, i, 0:16] = jnp.zeros((16,), jnp.float32)
        mbuf[0, i, 16:32] = jnp.zeros((16,), jnp.float32)
        return 0
    lax.fori_loop(0, 128, zb, 0)

    def zc(k, _):
        pltpu.sync_copy(zsrc, acc.at[pl.ds(s * RPT + k * 128, 128)])
        return 0
    lax.fori_loop(0, 24, zc, 0)
    pltpu.sync_copy(zsrc.at[pl.ds(0, 56)],
                    acc.at[pl.ds(s * RPT + 24 * 128, 56)])

    @pl.when(s == 15)
    def _():
        pltpu.sync_copy(zsrc.at[pl.ds(0, 80)], acc.at[pl.ds(16 * RPT, 80)])
    plsc.subcore_barrier()

    def wait_slot(slot):
        pltpu.make_async_copy(u_hbm.at[ibufr.at[0, 0]], mbuf.at[slot],
                              sems[slot]).wait()

    def load_idx(b, par):
        pltpu.sync_copy(
            rows2_hbm.at[pl.ds(c * NSUB + s * 400 + b * IB, IB)],
            ibufr.at[par])
        pltpu.sync_copy(col_hbm.at[pl.ds(s * 400 + b * IB, IB)],
                        ibufc.at[par])

    def gather(slot, par, j):
        pltpu.async_copy(u_hbm.at[ibufr.at[par, j]], mbuf.at[slot],
                         sems[slot])

    def scatter(slot, par, j):
        pltpu.async_copy(mbuf.at[slot], acc.at[ibufc.at[par, j]],
                         sems[slot], add=True)

    def batch_body(b, p):
        for j in range(IB):
            slot = j % NSLOT
            if j < NSLOT:
                @pl.when(b >= 1)
                def _():
                    wait_slot(slot)
            else:
                wait_slot(slot)
            gather(slot, p, j)
            if j == NSLOT:
                @pl.when(b < NB - 1)
                def _():
                    load_idx(b + 1, 1 - p)
            jb = j - LAG
            slotb = jb % NSLOT
            if jb < 0:
                @pl.when(b >= 1)
                def _():
                    _seq = (wait_slot(slotb), scatter(slotb, 1 - p, jb + IB))
            else:
                wait_slot(slotb)
                scatter(slotb, p, jb)

    load_idx(jnp.int32(0), 0)

    def outer(bb, _):
        batch_body(2 * bb, 0)
        batch_body(2 * bb + 1, 1)
        return 0
    lax.fori_loop(0, NB // 2, outer, 0)

    for j in range(IB - LAG, IB):
        slot = j % NSLOT
        wait_slot(slot)
        scatter(slot, 1, j)
    for slot in range(NSLOT):
        wait_slot(slot)
    plsc.subcore_barrier()

    bounce = mbuf.at[0]

    def cp(k, _):
        pltpu.sync_copy(acc.at[pl.ds(s * RPT + k * 128, 128)], bounce)
        pltpu.sync_copy(bounce,
                        s_hbm.at[pl.ds(c * ACC_D + s * RPT + k * 128, 128)])
        return 0
    lax.fori_loop(0, 24, cp, 0)

    @pl.when(s < 15)
    def _():
        pltpu.sync_copy(acc.at[pl.ds(s * RPT + 3072, 56)],
                        bounce.at[pl.ds(0, 56)])
        pltpu.sync_copy(bounce.at[pl.ds(0, 56)],
                        s_hbm.at[pl.ds(c * ACC_D + s * RPT + 3072, 56)])

    @pl.when(s == 15)
    def _():
        pltpu.sync_copy(acc.at[pl.ds(15 * RPT + 3072, 8)],
                        bounce.at[pl.ds(0, 8)])
        pltpu.sync_copy(bounce.at[pl.ds(0, 8)],
                        s_hbm.at[pl.ds(c * ACC_D + 15 * RPT + 3072, 8)])


_prop_call = functools.partial(
    pl.kernel,
    out_type=jax.ShapeDtypeStruct((2 * ACC_D, H), jnp.float32),
    mesh=_mesh,
    scratch_types=[
        pltpu.VMEM_SHARED((ACC_ROWS, H), jnp.float32),
        pltpu.VMEM((2, IB, 128), jnp.int32),
        pltpu.VMEM((2, IB, 128), jnp.int32),
        pltpu.VMEM((NSLOT, 128, H), jnp.float32),
        pltpu.SemaphoreType.DMA,
        pltpu.SemaphoreType.DMA,
        pltpu.SemaphoreType.DMA,
        pltpu.SemaphoreType.DMA,
        pltpu.SemaphoreType.DMA,
    ],
    compiler_params=_sc_params,
)(_prop_body)


HROWS = ACC_D
UPACK = HROWS // 4


def _prep_kernel(row_ref, out_ref):
    c = pl.program_id(0)
    v = row_ref[...]
    out_ref[...] = jnp.where(v >= N, 0, v + c * HROWS)


def _prep_call(rowp):
    return pl.pallas_call(
        _prep_kernel,
        grid=(2, NSUB // 128),
        in_specs=[pl.BlockSpec((128, 128), lambda c, i: (i, 0))],
        out_specs=pl.BlockSpec((128, 128), lambda c, i: (c * (NSUB // 128) + i, 0)),
        out_shape=jax.ShapeDtypeStruct((2 * NSUB // 128 * 128, 128), jnp.int32),
    )(rowp)


def _dis_of(ca_ref, cb_ref):
    deg = ca_ref[0] + cb_ref[0] + 1.0
    return lax.transpose(lax.rsqrt(deg), (1, 0))


_DN_T = (((0,), (0,)), ((), ()))


def _pack(p_ref, x):
    xc = lax.dot_general(p_ref[...], x, _DN_T,
                         preferred_element_type=jnp.float32)
    return jnp.concatenate(
        [xc[0:256], xc[256:512], xc[512:768], xc[768:1024]], axis=1)


def _unpack(p_ref, xp):
    xc = jnp.concatenate(
        [xp[:, 0:32], xp[:, 32:64], xp[:, 64:96], xp[:, 96:128]], axis=0)
    return jnp.dot(p_ref[...], xc, preferred_element_type=jnp.float32)


def _tc1_kernel(x_ref, f_ref, ca_ref, cb_ref, p_ref, wa_ref, wb_ref, ba_ref,
                bb_ref, u_ref):
    dis = _dis_of(ca_ref, cb_ref)
    x = x_ref[...]
    f = f_ref[...]
    dot = lambda a, b: lax.dot_general(a, b, _DN_T,
                                       preferred_element_type=jnp.float32)
    ta = dot(x, wa_ref[0:64, :]) + dot(f, wa_ref[64:128, :]) + ba_ref[...]
    tb = dot(x, wb_ref[0:64, :]) + dot(f, wb_ref[64:128, :]) + bb_ref[...]
    u_ref[0] = _pack(p_ref, dis * ta)
    u_ref[1] = _pack(p_ref, dis * tb)


_CNTA = pl.BlockSpec((1, 1, BLK), lambda i: (i, 0, 0))
_CNTB = pl.BlockSpec((1, 1, BLK), lambda i: (ACC_D // BLK + i, 0, 0))


def _tc1_call(x, feat, cnt4, pm, wa, wb, ba, bb):
    full = lambda shape: pl.BlockSpec(shape, lambda i: tuple(0 for _ in shape))
    return pl.pallas_call(
        _tc1_kernel,
        grid=(GRID,),
        in_specs=[
            pl.BlockSpec((64, BLK), lambda i: (0, i)),
            pl.BlockSpec((64, BLK), lambda i: (0, i)),
            _CNTA, _CNTB,
            full((BLK, BLK)),
            full((128, H)), full((128, H)), full((1, H)), full((1, H)),
        ],
        out_specs=pl.BlockSpec((2, BLK // 4, 128), lambda i: (0, i, 0)),
        out_shape=jax.ShapeDtypeStruct((2, UPACK, 128), jnp.float32),
    )(x, feat, cnt4, cnt4, pm, wa, wb, ba, bb)


def _tc2_kernel(s_ref, u_ref, ca_ref, cb_ref, p_ref, wat_ref, wab_ref,
                wbt_ref, wbb_ref, ba_ref, bb_ref, uo_ref):
    dis = _dis_of(ca_ref, cb_ref)
    dis_p = _pack(p_ref, dis * jnp.ones((1, H), jnp.float32))
    ha = jnp.maximum(dis_p * (s_ref[0] + u_ref[0]), 0.0)
    hb = jnp.maximum(dis_p * (s_ref[1] + u_ref[1]), 0.0)
    mm = lambda a, w: jnp.dot(a, w[...], preferred_element_type=jnp.float32)
    ta = mm(ha, wat_ref) + mm(hb, wab_ref) + ba_ref[...]
    tb = mm(ha, wbt_ref) + mm(hb, wbb_ref) + bb_ref[...]
    uo_ref[0] = dis_p * ta
    uo_ref[1] = dis_p * tb


_SU = pl.BlockSpec((2, BLK // 4, 128), lambda i: (0, i, 0))


def _tc2_call(s1, u1, cnt4, pm, wat, wab, wbt, wbb, ba, bb):
    full = lambda shape: pl.BlockSpec(shape, lambda i: tuple(0 for _ in shape))
    return pl.pallas_call(
        _tc2_kernel,
        grid=(GRID,),
        in_specs=[
            _SU, _SU, _CNTA, _CNTB, full((BLK, BLK)),
            full((128, 128)), full((128, 128)), full((128, 128)),
            full((128, 128)), full((1, 128)), full((1, 128)),
        ],
        out_specs=pl.BlockSpec((2, BLK // 4, 128), lambda i: (0, i, 0)),
        out_shape=jax.ShapeDtypeStruct((2, UPACK, 128), jnp.float32),
    )(s1, u1, cnt4, cnt4, pm, wat, wab, wbt, wbb, ba, bb)


def _tc3_kernel(s_ref, u_ref, ca_ref, cb_ref, p_ref, w_ref, b_ref, o_ref):
    dis = _dis_of(ca_ref, cb_ref)
    ha = jnp.maximum(dis * _unpack(p_ref, s_ref[0] + u_ref[0]), 0.0)
    hb = jnp.maximum(dis * _unpack(p_ref, s_ref[1] + u_ref[1]), 0.0)
    t = (jnp.dot(ha, w_ref[0:H, :], preferred_element_type=jnp.float32)
         + jnp.dot(hb, w_ref[H:2 * H, :], preferred_element_type=jnp.float32)
         + b_ref[...])
    o_ref[...] = jnp.maximum(t, 0.0)


def _tc3_call(s2, u2, cnt4, pm, w, b):
    full = lambda shape: pl.BlockSpec(shape, lambda i: tuple(0 for _ in shape))
    return pl.pallas_call(
        _tc3_kernel,
        grid=(GRID,),
        in_specs=[
            _SU, _SU, _CNTA, _CNTB, full((BLK, BLK)),
            full((2 * H, 64)), full((1, 64)),
        ],
        out_specs=pl.BlockSpec((BLK, 64), lambda i: (i, 0)),
        out_shape=jax.ShapeDtypeStruct((N, 64), jnp.float32),
    )(s2, u2, cnt4, cnt4, pm, w, b)


def kernel(x, feat, edge_index, W1, b1, W2, b2, Wfc, bfc):
    row = edge_index[0]
    col = edge_index[1]
    pad = jnp.full((EPAD - E,), N, jnp.int32)
    rowp = jnp.concatenate([row, pad]).reshape(NSUB, 128)
    colp = jnp.concatenate([col, pad]).reshape(NSUB, 128)

    w1t = W1.T
    wa1, wb1 = w1t[:, :H], w1t[:, H:]
    ba1, bb1 = b1[:H].reshape(1, H), b1[H:].reshape(1, H)
    w2t = W2.T
    eye4 = jnp.eye(4, dtype=jnp.float32)
    bd = lambda m: jnp.kron(eye4, m)
    wat2 = bd(w2t[0:H, 0:H])
    wab2 = bd(w2t[H:2 * H, 0:H])
    wbt2 = bd(w2t[0:H, H:2 * H])
    wbb2 = bd(w2t[H:2 * H, H:2 * H])
    ba2 = jnp.tile(b2[:H], 4).reshape(1, 128)
    bb2 = jnp.tile(b2[H:], 4).reshape(1, 128)
    wfct = Wfc.T
    bfc2 = bfc.reshape(1, 64)
    narr = jnp.arange(BLK)
    pm = jnp.zeros((BLK, BLK), jnp.float32).at[
        narr, 256 * (narr % 4) + narr // 4].set(1.0)

    cnt = _deg_call(rowp)
    cnt4 = cnt.reshape(2 * ACC_D // BLK, 1, BLK)
    rows2 = _prep_call(rowp)

    u1 = _tc1_call(x.T, feat.T, cnt4, pm, wa1, wb1, ba1, bb1)
    s1 = _prop_call(u1.reshape(2 * ACC_D, H), rows2, colp)
    u2 = _tc2_call(s1.reshape(2, UPACK, 128), u1, cnt4, pm,
                   wat2, wab2, wbt2, wbb2, ba2, bb2)
    s2 = _prop_call(u2.reshape(2 * ACC_D, H), rows2, colp)
    return _tc3_call(s2.reshape(2, UPACK, 128), u2, cnt4, pm, wfct, bfc2)
